# baseline (speedup 1.0000x reference)
import jax
import jax.numpy as jnp
from jax import lax
from jax.experimental import pallas as pl
from jax.experimental.pallas import tpu as pltpu
from jax.experimental.pallas import tpu_sc as plsc

VOCAB = 1000000
EMBED = 64
SEQ = 200
BATCH = 4096

NC = 2
NS = 16
NW = NC * NS
BPW = BATCH // NW
L = 16
NEG = EMBED // L

ST = SEQ // 8
BT = BATCH // 128
ET = EMBED // 8


def _embed_kernel(x4_hbm, tab_hbm, pos_hbm, out_hbm,
                  idx_w, pos_v, gidx0, gidx1, rows0, rows1, tile0, tile1,
                  sem_g0, sem_g1, sem_w0, sem_w1):
    wid = lax.axis_index("s") * NC + lax.axis_index("c")

    pltpu.sync_copy(x4_hbm.at[:, wid], idx_w)
    pltpu.sync_copy(pos_hbm, pos_v)

    gidx = (gidx0, gidx1)
    rows = (rows0, rows1)
    tile = (tile0, tile1)
    sem_g = (sem_g0, sem_g1)
    sem_w = (sem_w0, sem_w1)

    def fire_gather(b, s):
        ts = s // 8
        ss = s % 8
        for g in range(128 // L):
            v = idx_w[ts, ss, pl.ds(g * L, L)]
            gidx[b][pl.ds(g * L, L)] = jnp.right_shift(v, 1)
        cp = pltpu.make_async_copy(tab_hbm.at[gidx[b]], rows[b], sem_g[b])
        cp.start()
        return cp

    def wb(b, s):
        return [
            pltpu.make_async_copy(
                tile[b].at[pl.ds(eb * 1024, 1024)],
                out_hbm.at[s, eb, wid],
                sem_w[b],
            )
            for eb in range(ET)
        ]

    stridec = jax.lax.iota(jnp.int32, L) * 128
    pv = [None] * NEG

    def transpose_rows(b, s):
        ts = s // 8
        ss = s % 8
        for g in range(NEG):
            pv[g] = pos_v[s, pl.ds(g * L, L)]

        def l_body(l, carry):
            tokv = idx_w[ts, ss, pl.ds(l * L, L)]
            for j in range(L):
                bl = l * L + j
                h = jnp.bitwise_and(tokv[j], 1) * EMBED
                for g in range(NEG):
                    vals = rows[b][bl, pl.ds(h + g * L, L)] + pv[g]
                    plsc.store_scatter(tile[b], [stridec + (2048 * g + bl)], vals)
            return carry

        lax.fori_loop(0, 128 // L, l_body, 0)

    cp = fire_gather(0, 0)
    cp.wait()
    fire_gather(1, 1)
    transpose_rows(0, 0)
    for c in wb(0, 0):
        c.start()

    def s_body(s2, carry):
        for b in range(2):
            s = 1 + 2 * s2 + b
            bb = (1 + b) % 2
            g1_desc = pltpu.make_async_copy(
                tab_hbm.at[gidx[bb]], rows[bb], sem_g[bb])
            g1_desc.wait()
            transpose_rows(bb, s)
            for c in wb(1 - bb, s - 1):
                c.wait()
            fire_gather(1 - bb, s + 1)
            for c in wb(bb, s):
                c.start()
        return carry

    lax.fori_loop(0, (SEQ - 2) // 2, s_body, 0)

    pltpu.make_async_copy(tab_hbm.at[gidx[1]], rows[1], sem_g[1]).wait()
    transpose_rows(1, SEQ - 1)
    for c in wb(0, SEQ - 2):
        c.wait()
    for c in wb(1, SEQ - 1):
        c.start()
    for c in wb(1, SEQ - 1):
        c.wait()


@jax.jit
def _embed(x4, tab2, position_table):
    mesh = plsc.VectorSubcoreMesh(core_axis_name="c", subcore_axis_name="s")
    return pl.kernel(
        _embed_kernel,
        mesh=mesh,
        out_type=jax.ShapeDtypeStruct((SEQ, ET, BT, 1024), jnp.float32),
        scratch_types=[
            pltpu.VMEM((ST, 8, 128), jnp.int32),
            pltpu.VMEM((SEQ, EMBED), jnp.float32),
            pltpu.VMEM((128,), jnp.int32),
            pltpu.VMEM((128,), jnp.int32),
            pltpu.VMEM((128, 128), jnp.float32),
            pltpu.VMEM((128, 128), jnp.float32),
            pltpu.VMEM((ET * 1024,), jnp.float32),
            pltpu.VMEM((ET * 1024,), jnp.float32),
            pltpu.SemaphoreType.DMA,
            pltpu.SemaphoreType.DMA,
            pltpu.SemaphoreType.DMA,
            pltpu.SemaphoreType.DMA,
        ],
        compiler_params=pltpu.CompilerParams(
            use_tc_tiling_on_sc=False, needs_layout_passes=False),
    )(x4, tab2, position_table)


def kernel(x, token_table, position_table):
    x4 = x.reshape(BT, 128, ST, 8).transpose(2, 0, 3, 1)
    tab2 = token_table.reshape(VOCAB // 2, 128)
    out5 = _embed(x4, tab2, position_table).reshape(SEQ, ET, BT, 8, 128)
    return out5.transpose(2, 4, 0, 1, 3).reshape(BATCH, SEQ, EMBED)
